# Initial kernel scaffold; baseline (speedup 1.0000x reference)
import jax
import jax.numpy as jnp
from jax.experimental import pallas as pl


def kernel(h, edge_index, params):
    raise NotImplementedError("write your pallas kernel here")


# speedup vs baseline: 9.6090x; 9.6090x over previous
import functools

import jax
import jax.numpy as jnp
from jax import lax
from jax.experimental import pallas as pl
from jax.experimental.pallas import tpu as pltpu
from jax.experimental.pallas import tpu_sc as plsc

N_NODES = 10000
N_EDGES = 320000
HID = 32
BN_EPS = 1e-5

NC = 2
NS = 16
NW = NC * NS
EBLK = 80
NBLK_TOT = N_EDGES // EBLK
NBLK_W = NBLK_TOT // NW
ROWS_SUB = 640
N_PAD = ROWS_SUB * NS

def _sc_body(z_hbm, src_hbm, dst_hbm, out_hbm,
             src_v, dst_v, rows_v, stage_v, agg_sh, sem):
    c = lax.axis_index("c")
    s = lax.axis_index("s")
    wid = c * NS + s

    zeros16 = jnp.zeros((16,), jnp.float32)

    def _zero(i, carry):
        stage_v[i, pl.ds(0, 16)] = zeros16
        stage_v[i, pl.ds(16, 16)] = zeros16
        return carry

    lax.fori_loop(0, ROWS_SUB, _zero, 0)
    pltpu.sync_copy(stage_v, agg_sh.at[pl.ds(s * ROWS_SUB, ROWS_SUB)])

    pltpu.sync_copy(src_hbm.at[wid], src_v)
    pltpu.sync_copy(dst_hbm.at[wid], dst_v)
    plsc.subcore_barrier()

    def _block(j, carry):
        pltpu.async_copy(z_hbm.at[src_v.at[j]], rows_v, sem).wait()
        pltpu.sync_copy(rows_v, agg_sh.at[dst_v.at[j]], add=True)
        return carry

    lax.fori_loop(0, NBLK_W, _block, 0)
    plsc.subcore_barrier()

    pltpu.sync_copy(agg_sh.at[pl.ds(s * ROWS_SUB, ROWS_SUB)],
                    out_hbm.at[c, pl.ds(s * ROWS_SUB, ROWS_SUB)])


@functools.cache
def _get_sc_kernel():
    mesh = plsc.VectorSubcoreMesh(core_axis_name="c", subcore_axis_name="s")
    return pl.kernel(
        _sc_body,
        out_type=jax.ShapeDtypeStruct((NC, N_PAD, HID), jnp.float32),
        mesh=mesh,
        scratch_types=[
            pltpu.VMEM((NBLK_W, EBLK), jnp.int32),
            pltpu.VMEM((NBLK_W, EBLK), jnp.int32),
            pltpu.VMEM((EBLK, HID), jnp.float32),
            pltpu.VMEM((ROWS_SUB, HID), jnp.float32),
            pltpu.VMEM_SHARED((N_PAD, HID), jnp.float32),
            pltpu.SemaphoreType.DMA,
        ],
        compiler_params=pltpu.CompilerParams(use_tc_tiling_on_sc=False),
    )


def _sc_segment_sum(z, src2, dst2):
    return _get_sc_kernel()(z, src2, dst2)


def _bn(x, gamma, beta):
    mu = jnp.mean(x, axis=0)
    var = jnp.mean((x - mu) ** 2, axis=0)
    return gamma * (x - mu) * lax.rsqrt(var + BN_EPS) + beta


def _tc_head_body(h_ref, w1_ref, wp_ref, bp_ref, z_ref, s_ref):
    hmat = h_ref[...]
    z_ref[...] = jnp.dot(hmat, w1_ref[...], preferred_element_type=jnp.float32)
    pooled = jnp.sum(hmat, axis=0, keepdims=True)
    s_ref[...] = (jnp.dot(pooled, wp_ref[...],
                          preferred_element_type=jnp.float32) + bp_ref[...])


def _tc_layer_body(*refs, has_next):
    (z_ref, agg_ref, b0,
     w1, b1, w2, b2, w3, b3, w4, b4,
     g0, e0, g1, e1, g2, e2, g3, e3,
     ga, ea, go, eo) = refs[:23]
    rest = refs[23:]
    if has_next:
        wn, wp, bp, zn_ref, s_ref = rest
    else:
        wp, bp, s_ref = rest

    y = z_ref[...] + agg_ref[0, :N_NODES] + agg_ref[1, :N_NODES] + b0[...]
    y = jax.nn.relu(_bn(y, g0[...], e0[...]))
    for w, bv, g, e in ((w1, b1, g1, e1), (w2, b2, g2, e2), (w3, b3, g3, e3)):
        y = jnp.dot(y, w[...], preferred_element_type=jnp.float32) + bv[...]
        y = jax.nn.relu(_bn(y, g[...], e[...]))
    y = jnp.dot(y, w4[...], preferred_element_type=jnp.float32) + b4[...]
    x = jax.nn.relu(_bn(y, ga[...], ea[...]))
    x = jax.nn.relu(_bn(x, go[...], eo[...]))
    if has_next:
        zn_ref[...] = jnp.dot(x, wn[...], preferred_element_type=jnp.float32)
    pooled = jnp.sum(x, axis=0, keepdims=True)
    s_ref[...] = (jnp.dot(pooled, wp[...],
                          preferred_element_type=jnp.float32) + bp[...])


def _apply_layer(z, aggp, lp, w1_next, wp, bp):
    lins = lp["mlp"]["lins"]
    bns = lp["mlp"]["bns"]
    args = [z, aggp, lins[0][1]]
    for i in range(1, 5):
        args += [lins[i][0], lins[i][1]]
    for i in range(4):
        args += [bns[i][0], bns[i][1]]
    args += [lp["bn_apply"][0], lp["bn_apply"][1],
             lp["bn_outer"][0], lp["bn_outer"][1]]
    has_next = w1_next is not None
    if has_next:
        args.append(w1_next)
    args += [wp, bp]
    out_shape = [jax.ShapeDtypeStruct((1, 16), jnp.float32)]
    if has_next:
        out_shape = [jax.ShapeDtypeStruct((N_NODES, HID), jnp.float32)] + out_shape
    return pl.pallas_call(
        functools.partial(_tc_layer_body, has_next=has_next),
        out_shape=out_shape,
    )(*args)


def kernel(h, edge_index, params):
    src2 = edge_index[0].reshape(NW, NBLK_W, EBLK)
    dst2 = edge_index[1].reshape(NW, NBLK_W, EBLK)
    gin = params["gin"]
    pred = params["pred"]

    z, score = pl.pallas_call(
        _tc_head_body,
        out_shape=[jax.ShapeDtypeStruct((N_NODES, HID), jnp.float32),
                   jax.ShapeDtypeStruct((1, 16), jnp.float32)],
    )(h, gin[0]["mlp"]["lins"][0][0], pred[0][0], pred[0][1])

    for l in range(3):
        aggp = _sc_segment_sum(z, src2, dst2)
        w1_next = gin[l + 1]["mlp"]["lins"][0][0] if l < 2 else None
        outs = _apply_layer(z, aggp, gin[l], w1_next,
                            pred[l + 1][0], pred[l + 1][1])
        if l < 2:
            z, s = outs
        else:
            (s,) = outs
        score = score + s
    return score


# speedup vs baseline: 17.5544x; 1.8269x over previous
import functools

import jax
import jax.numpy as jnp
from jax import lax
from jax.experimental import pallas as pl
from jax.experimental.pallas import tpu as pltpu
from jax.experimental.pallas import tpu_sc as plsc

N_NODES = 10000
N_EDGES = 320000
HID = 32
BN_EPS = 1e-5

NC = 2
NS = 16
NW = NC * NS
EBLK = 80
NBLK_TOT = N_EDGES // EBLK
NBLK_W = NBLK_TOT // NW
ROWS_SUB = 640
N_PAD = ROWS_SUB * NS
NBUF = 5

def _sc_body(z_hbm, src_hbm, dst_hbm, out_hbm,
             src_v, dst_v, rows_v, stage_v, agg_sh, gsem, ssem):
    c = lax.axis_index("c")
    s = lax.axis_index("s")
    wid = c * NS + s

    zeros16 = jnp.zeros((16,), jnp.float32)

    def _zero(i, carry):
        stage_v[i, pl.ds(0, 16)] = zeros16
        stage_v[i, pl.ds(16, 16)] = zeros16
        return carry

    lax.fori_loop(0, ROWS_SUB, _zero, 0)
    pltpu.sync_copy(stage_v, agg_sh.at[pl.ds(s * ROWS_SUB, ROWS_SUB)])

    pltpu.sync_copy(src_hbm.at[wid], src_v)
    pltpu.sync_copy(dst_hbm.at[wid], dst_v)
    plsc.subcore_barrier()

    for b in range(NBUF):
        pltpu.async_copy(z_hbm.at[src_v.at[b]], rows_v.at[b], gsem.at[b])

    def _group(g, carry):
        j0 = g * NBUF
        descs = []
        for b in range(NBUF):
            j = j0 + b
            pltpu.make_async_copy(z_hbm.at[src_v.at[j]], rows_v.at[b],
                                  gsem.at[b]).wait()
            descs.append(pltpu.async_copy(rows_v.at[b],
                                          agg_sh.at[dst_v.at[j]],
                                          ssem.at[b], add=True))
        for b in range(NBUF):
            j = j0 + b
            descs[b].wait()

            @pl.when(j + NBUF < NBLK_W)
            def _():
                pltpu.async_copy(z_hbm.at[src_v.at[j + NBUF]], rows_v.at[b],
                                 gsem.at[b])
        return carry

    lax.fori_loop(0, NBLK_W // NBUF, _group, 0)
    plsc.subcore_barrier()

    pltpu.sync_copy(agg_sh.at[pl.ds(s * ROWS_SUB, ROWS_SUB)],
                    out_hbm.at[c, pl.ds(s * ROWS_SUB, ROWS_SUB)])


@functools.cache
def _get_sc_kernel():
    mesh = plsc.VectorSubcoreMesh(core_axis_name="c", subcore_axis_name="s")
    return pl.kernel(
        _sc_body,
        out_type=jax.ShapeDtypeStruct((NC, N_PAD, HID), jnp.float32),
        mesh=mesh,
        scratch_types=[
            pltpu.VMEM((NBLK_W, EBLK), jnp.int32),
            pltpu.VMEM((NBLK_W, EBLK), jnp.int32),
            pltpu.VMEM((NBUF, EBLK, HID), jnp.float32),
            pltpu.VMEM((ROWS_SUB, HID), jnp.float32),
            pltpu.VMEM_SHARED((N_PAD, HID), jnp.float32),
            pltpu.SemaphoreType.DMA((NBUF,)),
            pltpu.SemaphoreType.DMA((NBUF,)),
        ],
        compiler_params=pltpu.CompilerParams(use_tc_tiling_on_sc=False),
    )


def _sc_segment_sum(z, src2, dst2):
    return _get_sc_kernel()(z, src2, dst2)


def _bn(x, gamma, beta):
    mu = jnp.mean(x, axis=0)
    var = jnp.mean((x - mu) ** 2, axis=0)
    return gamma * (x - mu) * lax.rsqrt(var + BN_EPS) + beta


def _tc_head_body(h_ref, w1_ref, wp_ref, bp_ref, z_ref, s_ref):
    hmat = h_ref[...]
    z_ref[...] = jnp.dot(hmat, w1_ref[...], preferred_element_type=jnp.float32)
    pooled = jnp.sum(hmat, axis=0, keepdims=True)
    s_ref[...] = (jnp.dot(pooled, wp_ref[...],
                          preferred_element_type=jnp.float32) + bp_ref[...])


def _tc_layer_body(*refs, has_next):
    (z_ref, agg_ref, b0,
     w1, b1, w2, b2, w3, b3, w4, b4,
     g0, e0, g1, e1, g2, e2, g3, e3,
     ga, ea, go, eo) = refs[:23]
    rest = refs[23:]
    if has_next:
        wn, wp, bp, zn_ref, s_ref = rest
    else:
        wp, bp, s_ref = rest

    y = z_ref[...] + agg_ref[0, :N_NODES] + agg_ref[1, :N_NODES] + b0[...]
    y = jax.nn.relu(_bn(y, g0[...], e0[...]))
    for w, bv, g, e in ((w1, b1, g1, e1), (w2, b2, g2, e2), (w3, b3, g3, e3)):
        y = jnp.dot(y, w[...], preferred_element_type=jnp.float32) + bv[...]
        y = jax.nn.relu(_bn(y, g[...], e[...]))
    y = jnp.dot(y, w4[...], preferred_element_type=jnp.float32) + b4[...]
    x = jax.nn.relu(_bn(y, ga[...], ea[...]))
    x = jax.nn.relu(_bn(x, go[...], eo[...]))
    if has_next:
        zn_ref[...] = jnp.dot(x, wn[...], preferred_element_type=jnp.float32)
    pooled = jnp.sum(x, axis=0, keepdims=True)
    s_ref[...] = (jnp.dot(pooled, wp[...],
                          preferred_element_type=jnp.float32) + bp[...])


def _apply_layer(z, aggp, lp, w1_next, wp, bp):
    lins = lp["mlp"]["lins"]
    bns = lp["mlp"]["bns"]
    args = [z, aggp, lins[0][1]]
    for i in range(1, 5):
        args += [lins[i][0], lins[i][1]]
    for i in range(4):
        args += [bns[i][0], bns[i][1]]
    args += [lp["bn_apply"][0], lp["bn_apply"][1],
             lp["bn_outer"][0], lp["bn_outer"][1]]
    has_next = w1_next is not None
    if has_next:
        args.append(w1_next)
    args += [wp, bp]
    out_shape = [jax.ShapeDtypeStruct((1, 16), jnp.float32)]
    if has_next:
        out_shape = [jax.ShapeDtypeStruct((N_NODES, HID), jnp.float32)] + out_shape
    return pl.pallas_call(
        functools.partial(_tc_layer_body, has_next=has_next),
        out_shape=out_shape,
    )(*args)


def kernel(h, edge_index, params):
    src2 = edge_index[0].reshape(NW, NBLK_W, EBLK)
    dst2 = edge_index[1].reshape(NW, NBLK_W, EBLK)
    gin = params["gin"]
    pred = params["pred"]

    z, score = pl.pallas_call(
        _tc_head_body,
        out_shape=[jax.ShapeDtypeStruct((N_NODES, HID), jnp.float32),
                   jax.ShapeDtypeStruct((1, 16), jnp.float32)],
    )(h, gin[0]["mlp"]["lins"][0][0], pred[0][0], pred[0][1])

    for l in range(3):
        aggp = _sc_segment_sum(z, src2, dst2)
        w1_next = gin[l + 1]["mlp"]["lins"][0][0] if l < 2 else None
        outs = _apply_layer(z, aggp, gin[l], w1_next,
                            pred[l + 1][0], pred[l + 1][1])
        if l < 2:
            z, s = outs
        else:
            (s,) = outs
        score = score + s
    return score
